# baseline (speedup 1.0000x reference)
import jax
import jax.numpy as jnp
from jax import lax
from jax.experimental import pallas as pl
from jax.experimental.pallas import tpu as pltpu
from jax.experimental.pallas import tpu_sc as plsc

NUM_CORES = 2
NUM_SUBCORES = 16
NUM_WORKERS = NUM_CORES * NUM_SUBCORES
LANES = 16
D = 64
ROWS_PER_TILE = 31250
MAX_CHUNK = 125
B_PER_TILE = 128
SEQ = 50


def _wid():
    return lax.axis_index("s") * NUM_CORES + lax.axis_index("c")


def _max_body(table_hbm, part_hbm, buf0, buf1, ramp0, ramp1, acc_v, s0, s1):
    wid = _wid()
    base = wid * ROWS_PER_TILE
    iota = lax.broadcasted_iota(jnp.int32, (LANES,), 0)

    def chunk_start(j, buf, ramp, sem):
        first = base + j * MAX_CHUNK
        for c in range(MAX_CHUNK // LANES + 1):
            ramp[pl.ds(c * LANES, LANES)] = first + c * LANES + iota
        pltpu.async_copy(table_hbm.at[ramp.at[pl.ds(0, MAX_CHUNK)]], buf, sem)

    def chunk_reduce(buf, acc):
        def row_body(r, a):
            for c in range(D // LANES):
                a = jnp.maximum(a, jnp.abs(buf[r, pl.ds(c * LANES, LANES)]))
            return a

        return lax.fori_loop(0, MAX_CHUNK, row_body, acc, unroll=4)

    def chunk_wait(buf, ramp, sem):
        pltpu.make_async_copy(
            table_hbm.at[ramp.at[pl.ds(0, MAX_CHUNK)]], buf, sem).wait()

    chunk_start(0, buf0, ramp0, s0)
    chunk_start(1, buf1, ramp1, s1)
    n_pairs = ROWS_PER_TILE // MAX_CHUNK // 2

    def body(t, acc):
        chunk_wait(buf0, ramp0, s0)
        acc = chunk_reduce(buf0, acc)

        @pl.when(t < n_pairs - 1)
        def _():
            chunk_start(2 * t + 2, buf0, ramp0, s0)

        chunk_wait(buf1, ramp1, s1)
        acc = chunk_reduce(buf1, acc)

        @pl.when(t < n_pairs - 1)
        def _():
            chunk_start(2 * t + 3, buf1, ramp1, s1)

        return acc

    acc = lax.fori_loop(0, n_pairs, body, jnp.zeros((LANES,), jnp.float32))
    acc_v[...] = acc
    pltpu.sync_copy(acc_v, part_hbm.at[wid])


def _max_partials(weight):
    mesh = plsc.VectorSubcoreMesh(core_axis_name="c", subcore_axis_name="s")
    f = pl.kernel(
        _max_body,
        out_type=jax.ShapeDtypeStruct((NUM_WORKERS, LANES), jnp.float32),
        mesh=mesh,
        scratch_types=[
            pltpu.VMEM((MAX_CHUNK, D), jnp.float32),
            pltpu.VMEM((MAX_CHUNK, D), jnp.float32),
            pltpu.VMEM((128,), jnp.int32),
            pltpu.VMEM((128,), jnp.int32),
            pltpu.VMEM((LANES,), jnp.float32),
            pltpu.SemaphoreType.DMA,
            pltpu.SemaphoreType.DMA,
        ],
        compiler_params=pltpu.CompilerParams(use_tc_tiling_on_sc=False),
    )
    return f(weight)


def _gather_body(idx_hbm, table_hbm, maxv_hbm, out_hbm,
                 idx_v, maxv_v, rows0, rows1, out0, out1,
                 g0, g1, o0, o1):
    wid = _wid()
    b0 = wid * B_PER_TILE

    pltpu.sync_copy(idx_hbm.at[wid], idx_v)
    pltpu.sync_copy(maxv_hbm, maxv_v)

    vmax = maxv_v[...]
    vneg = -vmax
    vthr = vmax * (2.0 ** -24)

    def quantize(rows_v, out_v):
        def row_body(r, carry):
            for c in range(D // LANES):
                w = rows_v[r, pl.ds(c * LANES, LANES)]
                out_v[r, pl.ds(c * LANES, LANES)] = jnp.where(
                    w > vthr, vmax, vneg)
            return carry

        lax.fori_loop(0, SEQ, row_body, 0, unroll=2)

    pltpu.async_copy(table_hbm.at[idx_v.at[0]], rows0, g0)
    pltpu.async_copy(table_hbm.at[idx_v.at[1]], rows1, g1)
    n_pairs = B_PER_TILE // 2

    def body(t, carry):
        pltpu.make_async_copy(table_hbm.at[idx_v.at[2 * t]], rows0, g0).wait()

        @pl.when(t > 0)
        def _():
            pltpu.make_async_copy(out0, out_hbm.at[b0], o0).wait()

        quantize(rows0, out0)
        pltpu.async_copy(out0, out_hbm.at[b0 + 2 * t], o0)

        @pl.when(t < n_pairs - 1)
        def _():
            pltpu.async_copy(table_hbm.at[idx_v.at[2 * t + 2]], rows0, g0)

        pltpu.make_async_copy(
            table_hbm.at[idx_v.at[2 * t + 1]], rows1, g1).wait()

        @pl.when(t > 0)
        def _():
            pltpu.make_async_copy(out1, out_hbm.at[b0], o1).wait()

        quantize(rows1, out1)
        pltpu.async_copy(out1, out_hbm.at[b0 + 2 * t + 1], o1)

        @pl.when(t < n_pairs - 1)
        def _():
            pltpu.async_copy(table_hbm.at[idx_v.at[2 * t + 3]], rows1, g1)

        return carry

    lax.fori_loop(0, n_pairs, body, 0)
    pltpu.make_async_copy(out0, out_hbm.at[b0], o0).wait()
    pltpu.make_async_copy(out1, out_hbm.at[b0], o1).wait()


def _gather_quant(idx3, weight, maxvec):
    b, s = NUM_WORKERS * B_PER_TILE, SEQ
    mesh = plsc.VectorSubcoreMesh(core_axis_name="c", subcore_axis_name="s")
    f = pl.kernel(
        _gather_body,
        out_type=jax.ShapeDtypeStruct((b, s, D), jnp.float32),
        mesh=mesh,
        scratch_types=[
            pltpu.VMEM((B_PER_TILE, SEQ), jnp.int32),
            pltpu.VMEM((LANES,), jnp.float32),
            pltpu.VMEM((SEQ, D), jnp.float32),
            pltpu.VMEM((SEQ, D), jnp.float32),
            pltpu.VMEM((SEQ, D), jnp.float32),
            pltpu.VMEM((SEQ, D), jnp.float32),
            pltpu.SemaphoreType.DMA,
            pltpu.SemaphoreType.DMA,
            pltpu.SemaphoreType.DMA,
            pltpu.SemaphoreType.DMA,
        ],
        compiler_params=pltpu.CompilerParams(use_tc_tiling_on_sc=False),
    )
    return f(idx3, weight, maxvec)


def kernel(input, weight):
    b, s = input.shape
    assert b == NUM_WORKERS * B_PER_TILE and s == SEQ
    idx3 = input.astype(jnp.int32).reshape(NUM_WORKERS, B_PER_TILE, SEQ)
    partials = _max_partials(weight)
    maxvec = jnp.broadcast_to(jnp.max(partials), (LANES,))
    return _gather_quant(idx3, weight, maxvec)
